# baseline (speedup 1.0000x reference)
import functools

import jax
import jax.numpy as jnp
from jax import lax
from jax.experimental import pallas as pl
from jax.experimental.pallas import tpu as pltpu
from jax.experimental.pallas import tpu_sc as plsc

_TABLE_ROWS = (30, 10, 3, 256, 4, 9, 13, 31, 10)
_TABLE_DIMS = (16, 16, 8, 32, 8, 16, 8, 16, 8)
_NT = len(_TABLE_DIMS)

_SC_TABLES = (3,)
_TC_TABLES = tuple(k for k in range(_NT) if k not in _SC_TABLES)

_B, _L = 4096, 200
_N = _B * _L

_INFO = plsc.get_sparse_core_info()
_NC, _NS = _INFO.num_cores, _INFO.num_subcores
_NW = _NC * _NS
_TOK_PER_W = _N // _NW
_C = 256
_NCH = _TOK_PER_W // _C


def _make_sc_call():
    mesh = plsc.VectorSubcoreMesh(core_axis_name="c", subcore_axis_name="s")
    sc_rows = [_TABLE_ROWS[k] for k in _SC_TABLES]
    sc_dims = [_TABLE_DIMS[k] for k in _SC_TABLES]
    nsc = len(_SC_TABLES)
    out_type = [
        jax.ShapeDtypeStruct((_N // 2, 16) if d == 8 else (_N, d), jnp.float32)
        for d in sc_dims
    ]
    scratch = []
    for n, d in zip(sc_rows, sc_dims):
        if d == 8:
            scratch.append(pltpu.VMEM((n * 8 + 24,), jnp.float32))
            scratch.append(pltpu.VMEM((n * n * 16 + 16,), jnp.float32))
        else:
            scratch.append(pltpu.VMEM((n * d,), jnp.float32))
    scratch += [pltpu.VMEM((_C * _NT,), jnp.int32) for _ in range(2)]
    scratch += [
        pltpu.VMEM((_C // 2, 16) if d == 8 else (_C, d), jnp.float32)
        for _ in range(2)
        for d in sc_dims
    ]
    scratch += [pltpu.SemaphoreType.DMA for _ in range(4)]

    @functools.partial(
        pl.kernel,
        out_type=out_type,
        mesh=mesh,
        scratch_types=scratch,
        compiler_params=pltpu.CompilerParams(
            needs_layout_passes=False, use_tc_tiling_on_sc=False
        ),
    )
    def sc_fn(*refs):
        it = iter(refs)
        x_hbm = next(it)
        w_hbm = [next(it) for _ in range(nsc)]
        outs_hbm = [next(it) for _ in range(nsc)]
        tabs, tab2 = [], []
        for d in sc_dims:
            tabs.append(next(it))
            tab2.append(next(it) if d == 8 else None)
        xv = [next(it) for _ in range(2)]
        obuf = [[next(it) for _ in range(nsc)] for _ in range(2)]
        xsem = [next(it) for _ in range(2)]
        osem = [next(it) for _ in range(2)]

        wid = lax.axis_index("s") * _NC + lax.axis_index("c")
        base0 = wid * _TOK_PER_W

        lanes = lax.iota(jnp.int32, 16)
        low8 = lanes < 8

        for j in range(nsc):
            n, d = sc_rows[j], sc_dims[j]
            if d != 8:
                pltpu.sync_copy(w_hbm[j], tabs[j])
                continue
            pltpu.sync_copy(w_hbm[j], tabs[j].at[pl.ds(8, n * 8)])

            def body_a(a, _, j=j, n=n):
                va = tabs[j][pl.ds(8 + a * 8, 16)]

                def body_b(b, __):
                    vb8 = tabs[j][pl.ds(b * 8, 16)]
                    comb = jnp.where(low8, va, vb8)
                    tab2[j][pl.ds((a * n + b) * 16, 16)] = comb
                    return __

                return lax.fori_loop(0, n, body_b, _)

            lax.fori_loop(0, n, body_a, 0)

        def x_copy(ci, s):
            return pltpu.make_async_copy(
                x_hbm.at[pl.ds((base0 + ci * _C) * _NT, _C * _NT)], xv[s], xsem[s]
            )

        def out_copy(ci, s, j):
            d = sc_dims[j]
            base = base0 + ci * _C
            if d == 8:
                dst = outs_hbm[j].at[pl.ds(base // 2, _C // 2)]
            else:
                dst = outs_hbm[j].at[pl.ds(base, _C)]
            return pltpu.make_async_copy(obuf[s][j], dst, osem[s])

        x_copy(0, 0).start()

        def process_chunk(ci, s, not_first):
            @pl.when(ci + 1 < _NCH)
            def _():
                x_copy(ci + 1, 1 - s).start()

            x_copy(ci, s).wait()

            @pl.when(not_first)
            def _():
                for j in range(nsc):
                    out_copy(ci, s, j).wait()

            @plsc.parallel_loop(0, _C // 16)
            def _(g):
                gs = g * 16
                tok9 = (gs + lanes) * _NT
                for j, k in enumerate(_SC_TABLES):
                    d = sc_dims[j]
                    xk = plsc.load_gather(xv[s], [tok9 + k])
                    if d == 8:
                        n = sc_rows[j]
                        for tt in range(0, 16, 2):
                            p = xk[tt] * n + xk[tt + 1]
                            row = tab2[j][pl.ds(p * 16, 16)]
                            obuf[s][j][g * 8 + tt // 2, :] = row
                    else:
                        for tt in range(16):
                            off = xk[tt] * d
                            for c in range(0, d, 16):
                                row = tabs[j][pl.ds(off + c, 16)]
                                obuf[s][j][gs + tt, pl.ds(c, 16)] = row

            for j in range(nsc):
                out_copy(ci, s, j).start()

        def pair_body(h, carry):
            process_chunk(2 * h, 0, h >= 1)
            process_chunk(2 * h + 1, 1, h >= 1)
            return carry

        lax.fori_loop(0, _NCH // 2, pair_body, 0)

        for s in range(2):
            for j in range(nsc):
                out_copy(0, s, j).wait()

    return sc_fn


_SC_CALL = _make_sc_call()

_BB = 16


def _tc_body(x_ref, *refs):
    w_refs = refs[: len(_TC_TABLES)]
    out_refs = refs[len(_TC_TABLES):]
    for bb in range(_BB):
        xs = x_ref[bb]
        for j, k in enumerate(_TC_TABLES):
            n = _TABLE_ROWS[k]
            onehot = (xs[:, k:k + 1] == lax.broadcasted_iota(jnp.int32, (_L, n), 1)
                      ).astype(jnp.float32)
            out_refs[j][bb] = jnp.dot(
                onehot, w_refs[j][...], preferred_element_type=jnp.float32
            )


def _make_tc_call():
    in_specs = [pl.BlockSpec((_BB, _L, _NT), lambda i: (i, 0, 0))]
    in_specs += [
        pl.BlockSpec((_TABLE_ROWS[k], _TABLE_DIMS[k]), lambda i: (0, 0))
        for k in _TC_TABLES
    ]
    out_specs = [
        pl.BlockSpec((_BB, _L, _TABLE_DIMS[k]), lambda i: (i, 0, 0))
        for k in _TC_TABLES
    ]
    out_shape = [
        jax.ShapeDtypeStruct((_B, _L, _TABLE_DIMS[k]), jnp.float32)
        for k in _TC_TABLES
    ]
    return pl.pallas_call(
        _tc_body,
        grid=(_B // _BB,),
        in_specs=in_specs,
        out_specs=out_specs,
        out_shape=out_shape,
    )


_TC_CALL = _make_tc_call()


def kernel(x, W_msg, W_act, W_finish, W_effect, W_phase, W_position, W_number,
           W_place, W_attrib):
    ws = (W_msg, W_act, W_finish, W_effect, W_phase, W_position, W_number,
          W_place, W_attrib)
    sc_outs = _SC_CALL(
        x.reshape(_N * _NT), *(ws[k].reshape(-1) for k in _SC_TABLES)
    )
    tc_outs = _TC_CALL(x, *(ws[k] for k in _TC_TABLES))
    result = [None] * _NT
    for j, k in enumerate(_SC_TABLES):
        result[k] = sc_outs[j].reshape(_B, _L, _TABLE_DIMS[k])
    for j, k in enumerate(_TC_TABLES):
        result[k] = tc_outs[j]
    return tuple(result)


# speedup vs baseline: 1.1727x; 1.1727x over previous
import functools

import jax
import jax.numpy as jnp
from jax import lax
from jax.experimental import pallas as pl
from jax.experimental.pallas import tpu as pltpu
from jax.experimental.pallas import tpu_sc as plsc

_TABLE_ROWS = (30, 10, 3, 256, 4, 9, 13, 31, 10)
_TABLE_DIMS = (16, 16, 8, 32, 8, 16, 8, 16, 8)
_NT = len(_TABLE_DIMS)

_B, _L = 4096, 200
_N = _B * _L

_INFO = plsc.get_sparse_core_info()
_NC, _NS = _INFO.num_cores, _INFO.num_subcores
_NW = _NC * _NS
_TOK_PER_W = _N // _NW
_C = 400
_NCH = _TOK_PER_W // _C


def _make_sc_call():
    mesh = plsc.VectorSubcoreMesh(core_axis_name="c", subcore_axis_name="s")
    out_type = [
        jax.ShapeDtypeStruct((_N // 2, 16), jnp.float32) if d == 8
        else jax.ShapeDtypeStruct((_B, _L, d), jnp.float32)
        for d in _TABLE_DIMS
    ]
    scratch = []
    for n, d in zip(_TABLE_ROWS, _TABLE_DIMS):
        if d == 8:
            scratch.append(pltpu.VMEM((n * 8 + 24,), jnp.float32))
            scratch.append(pltpu.VMEM((n * n * 16 + 16,), jnp.float32))
        else:
            scratch.append(pltpu.VMEM((n * d,), jnp.float32))
    scratch += [pltpu.VMEM((_C * _NT,), jnp.int32) for _ in range(2)]
    scratch += [
        pltpu.VMEM((_C // 2, 16) if d == 8 else (_C, d), jnp.float32)
        for _ in range(2)
        for d in _TABLE_DIMS
    ]
    scratch += [pltpu.SemaphoreType.DMA for _ in range(4)]

    @functools.partial(
        pl.kernel,
        out_type=out_type,
        mesh=mesh,
        scratch_types=scratch,
        compiler_params=pltpu.CompilerParams(
            needs_layout_passes=False, use_tc_tiling_on_sc=False
        ),
    )
    def sc_fn(*refs):
        it = iter(refs)
        x_hbm = next(it)
        w_hbm = [next(it) for _ in range(_NT)]
        outs_hbm = [next(it) for _ in range(_NT)]
        tabs, tab2 = [], []
        for d in _TABLE_DIMS:
            tabs.append(next(it))
            tab2.append(next(it) if d == 8 else None)
        xv = [next(it) for _ in range(2)]
        obuf = [[next(it) for _ in range(_NT)] for _ in range(2)]
        xsem = [next(it) for _ in range(2)]
        osem = [next(it) for _ in range(2)]

        wid = lax.axis_index("s") * _NC + lax.axis_index("c")
        base0 = wid * _TOK_PER_W
        base0b = wid * (_TOK_PER_W // _L)

        lanes = lax.iota(jnp.int32, 16)
        low8 = lanes < 8

        for k in range(_NT):
            n, d = _TABLE_ROWS[k], _TABLE_DIMS[k]
            if d != 8:
                pltpu.sync_copy(w_hbm[k], tabs[k])
                continue
            pltpu.sync_copy(w_hbm[k], tabs[k].at[pl.ds(8, n * 8)])

            def body_a(a, _, k=k, n=n):
                va = tabs[k][pl.ds(8 + a * 8, 16)]

                def body_b(b, __):
                    vb8 = tabs[k][pl.ds(b * 8, 16)]
                    comb = jnp.where(low8, va, vb8)
                    tab2[k][pl.ds((a * n + b) * 16, 16)] = comb
                    return __

                return lax.fori_loop(0, n, body_b, _)

            lax.fori_loop(0, n, body_a, 0)

        def x_copy(ci, s):
            return pltpu.make_async_copy(
                x_hbm.at[pl.ds((base0 + ci * _C) * _NT, _C * _NT)], xv[s], xsem[s]
            )

        def out_copies(ci, s, k):
            d = _TABLE_DIMS[k]
            if d == 8:
                base = base0 + ci * _C
                return [pltpu.make_async_copy(
                    obuf[s][k],
                    outs_hbm[k].at[pl.ds(base // 2, _C // 2)],
                    osem[s],
                )]
            b0 = base0b + ci * 2
            return [
                pltpu.make_async_copy(
                    obuf[s][k].at[pl.ds(p * _L, _L)],
                    outs_hbm[k].at[b0 + p],
                    osem[s],
                )
                for p in (0, 1)
            ]

        x_copy(0, 0).start()

        def process_chunk(ci, s, not_first):
            @pl.when(ci + 1 < _NCH)
            def _():
                x_copy(ci + 1, 1 - s).start()

            x_copy(ci, s).wait()

            @pl.when(not_first)
            def _():
                for k in range(_NT):
                    for cp in out_copies(ci, s, k):
                        cp.wait()

            @plsc.parallel_loop(0, _C // 16)
            def _(g):
                gs = g * 16
                tok9 = (gs + lanes) * _NT
                for k in range(_NT):
                    d = _TABLE_DIMS[k]
                    xk = plsc.load_gather(xv[s], [tok9 + k])
                    if d == 8:
                        n = _TABLE_ROWS[k]
                        for tt in range(0, 16, 2):
                            p = xk[tt] * n + xk[tt + 1]
                            row = tab2[k][pl.ds(p * 16, 16)]
                            obuf[s][k][g * 8 + tt // 2, :] = row
                    else:
                        for tt in range(16):
                            off = xk[tt] * d
                            for c in range(0, d, 16):
                                row = tabs[k][pl.ds(off + c, 16)]
                                obuf[s][k][gs + tt, pl.ds(c, 16)] = row

            for k in range(_NT):
                for cp in out_copies(ci, s, k):
                    cp.start()

        def pair_body(h, carry):
            process_chunk(2 * h, 0, h >= 1)
            process_chunk(2 * h + 1, 1, h >= 1)
            return carry

        lax.fori_loop(0, _NCH // 2, pair_body, 0)

        for s in range(2):
            for k in range(_NT):
                for cp in out_copies(0, s, k):
                    cp.wait()

    return sc_fn


_SC_CALL = _make_sc_call()


def kernel(x, W_msg, W_act, W_finish, W_effect, W_phase, W_position, W_number,
           W_place, W_attrib):
    ws = (W_msg, W_act, W_finish, W_effect, W_phase, W_position, W_number,
          W_place, W_attrib)
    outs = _SC_CALL(x.reshape(_N * _NT), *(w.reshape(-1) for w in ws))
    return tuple(
        o.reshape(_B, _L, d) if d == 8 else o
        for o, d in zip(outs, _TABLE_DIMS)
    )
